# baseline (speedup 1.0000x reference)
import functools

import jax
import jax.numpy as jnp
from jax import lax
from jax.experimental import pallas as pl
from jax.experimental.pallas import tpu as pltpu
from jax.experimental.pallas import tpu_sc as plsc

_R = 128
_N = 32768
_L = 16
_NC = 2
_NS = 16
_NW = _NC * _NS
_RPW = _R // _NW
_NV = _N // _L
_U = 8
_B = 11
_NB = 1 << _B
_NG = _NB // _L
_K = (_N - 1) // 2


def _scan_select(hist, cums, gcums, kk, nsub):
    zeros_i = jnp.zeros((_L,), jnp.int32)
    iota = lax.iota(jnp.int32, _L)

    @plsc.parallel_loop(0, _NG, unroll=4)
    def pa(g):
        h = hist[pl.ds(g * _L, _L)]
        hist[pl.ds(g * _L, _L)] = zeros_i
        for u in range(1, nsub):
            h = h + hist[pl.ds(u * _NB + g * _L, _L)]
            hist[pl.ds(u * _NB + g * _L, _L)] = zeros_i
        cums[pl.ds(g * _L, _L)] = plsc.cumsum(h)

    def pb(j, tot):
        gt = plsc.load_gather(cums, [(j * _L + iota) * _L + (_L - 1)])
        cs = plsc.cumsum(gt) + tot
        gcums[pl.ds(j * _L, _L)] = cs
        return jnp.max(cs)

    lax.fori_loop(0, _NG // _L, pb, jnp.int32(0))

    kt = kk + 1
    c1 = plsc.load_gather(gcums, [iota * 8 + 7])
    s1 = plsc.all_reduce_ffs(c1 >= kt)
    c2 = plsc.load_gather(gcums, [jnp.minimum(s1 * 8 + iota, _NG - 1)])
    s2 = plsc.all_reduce_ffs(c2 >= kt)
    g = s1 * 8 + s2
    gb = plsc.load_gather(gcums, [jnp.maximum(g - 1, 0)])
    base = jnp.where(g >= 1, gb, 0)
    cf = plsc.load_gather(cums, [g * _L + iota]) + base
    f = plsc.all_reduce_ffs(cf >= kt)
    b = g * _L + f
    lb = plsc.load_gather(cums, [jnp.maximum(b - 1, g * _L)])
    cum_before = jnp.where(f >= 1, lb + base, base)
    return b, kk - cum_before


def _pass1(row_v, hist):
    ones_i = jnp.ones((_L,), jnp.int32)

    @plsc.parallel_loop(0, _NV, unroll=_U, carry=jnp.zeros((_L,), jnp.float32))
    def p1(i, acc):
        v = row_v[pl.ds(i * _L, _L)]
        d1 = plsc.bitcast(v, jnp.int32) >> 19
        plsc.addupdate_scatter(hist, [((i & (_U - 1)) << _B) + d1], ones_i)
        return acc + v

    return jnp.sum(p1) * (1.0 / _N)


def _pass2(row_v, b1, hist):
    ones_i = jnp.ones((_L,), jnp.int32)

    @plsc.parallel_loop(0, _NV, unroll=_U)
    def p2(i):
        bits = plsc.bitcast(row_v[pl.ds(i * _L, _L)], jnp.int32)
        m = (bits >> 19) == b1
        plsc.addupdate_scatter(
            hist, [((i & (_U - 1)) << _B) + ((bits >> 8) & (_NB - 1))],
            ones_i, mask=m)


def _pass_fused(rowa_v, b1, hist2, rowb_v, hist1):
    ones_i = jnp.ones((_L,), jnp.int32)

    @plsc.parallel_loop(0, _NV, unroll=_U, carry=jnp.zeros((_L,), jnp.float32))
    def pf(i, acc):
        off = (i & (_U - 1)) << _B
        bits_a = plsc.bitcast(rowa_v[pl.ds(i * _L, _L)], jnp.int32)
        m = (bits_a >> 19) == b1
        plsc.addupdate_scatter(
            hist2, [off + ((bits_a >> 8) & (_NB - 1))], ones_i, mask=m)
        vb = rowb_v[pl.ds(i * _L, _L)]
        plsc.addupdate_scatter(
            hist1, [off + (plsc.bitcast(vb, jnp.int32) >> 19)], ones_i)
        return acc + vb

    return jnp.sum(pf) * (1.0 / _N)


def _sc_body(x_hbm, out_hbm, row_a, row_b, hist1, hist2, cums, gcums, out_v,
             sem_a, sem_b):
    wid = lax.axis_index("s") * _NC + lax.axis_index("c")
    iota = lax.iota(jnp.int32, _L)
    zeros_i = jnp.zeros((_L,), jnp.int32)

    rows = [row_a, row_b]
    sems = [sem_a, sem_b]
    base = wid * _RPW
    copies = [None, None]
    copies[0] = pltpu.async_copy(x_hbm.at[base], row_a, sem_a)

    @plsc.parallel_loop(0, _U * _NG, unroll=8)
    def z1(g):
        hist1[pl.ds(g * _L, _L)] = zeros_i

    @plsc.parallel_loop(0, _U * _NG, unroll=8)
    def z2(g):
        hist2[pl.ds(g * _L, _L)] = zeros_i

    def finish_row(j, b1, kk1, mean, out_acc):
        b2, _ = _scan_select(hist2, cums, gcums, kk1, _U)
        m_key = (b1 << 19) | (b2 << 8) | 128
        m_bit = plsc.bitcast(m_key, jnp.float32)
        margin = jnp.abs(m_bit - 0.5)
        delta = mean * margin
        rep = jnp.where(m_bit > 0.5, 0.5 + delta, m_bit + delta)
        return jnp.where(iota == j, rep, out_acc)

    copies[0].wait()
    copies[1] = pltpu.async_copy(x_hbm.at[base + 1], row_b, sem_b)
    means = [None] * _RPW
    means[0] = _pass1(row_a, hist1)
    kk0 = jnp.full((_L,), _K, jnp.int32)
    b1, kk1 = _scan_select(hist1, cums, gcums, kk0, _U)

    out_acc = jnp.zeros((_L,), jnp.float32)
    for j in range(_RPW - 1):
        copies[(j + 1) % 2].wait()
        means[j + 1] = _pass_fused(rows[j % 2], b1, hist2,
                                   rows[(j + 1) % 2], hist1)
        if j + 2 < _RPW:
            copies[j % 2] = pltpu.async_copy(
                x_hbm.at[base + j + 2], rows[j % 2], sems[j % 2])
        out_acc = finish_row(j, b1, kk1, means[j], out_acc)
        b1, kk1 = _scan_select(hist1, cums, gcums, kk0, _U)

    _pass2(rows[(_RPW - 1) % 2], b1, hist2)
    out_acc = finish_row(_RPW - 1, b1, kk1, means[_RPW - 1], out_acc)

    out_v[...] = out_acc
    pltpu.sync_copy(out_v, out_hbm.at[wid])


@functools.cache
def _build():
    mesh = plsc.VectorSubcoreMesh(core_axis_name="c", subcore_axis_name="s")
    return functools.partial(
        pl.kernel,
        out_type=jax.ShapeDtypeStruct((_NW, _L), jnp.float32),
        mesh=mesh,
        scratch_types=[
            pltpu.VMEM((_N,), jnp.float32),
            pltpu.VMEM((_N,), jnp.float32),
            pltpu.VMEM((_U * _NB,), jnp.int32),
            pltpu.VMEM((_U * _NB,), jnp.int32),
            pltpu.VMEM((_NB,), jnp.int32),
            pltpu.VMEM((_NG,), jnp.int32),
            pltpu.VMEM((_L,), jnp.float32),
            pltpu.SemaphoreType.DMA,
            pltpu.SemaphoreType.DMA,
        ],
        compiler_params=pltpu.CompilerParams(needs_layout_passes=False),
    )(_sc_body)


@jax.jit
def kernel(x):
    out2d = _build()(x)
    return out2d[:, :_RPW].reshape(-1)


# speedup vs baseline: 1.0210x; 1.0210x over previous
import functools

import jax
import jax.numpy as jnp
from jax import lax
from jax.experimental import pallas as pl
from jax.experimental.pallas import tpu as pltpu
from jax.experimental.pallas import tpu_sc as plsc

_R = 128
_N = 32768
_L = 16
_NC = 2
_NS = 16
_NW = _NC * _NS
_RPW = _R // _NW
_NV = _N // _L
_U = 8
_B = 11
_NB = 1 << _B
_NG = _NB // _L
_K = (_N - 1) // 2


def _scan_select(hist, cums, gcums, kk, nsub):
    zeros_i = jnp.zeros((_L,), jnp.int32)
    iota = lax.iota(jnp.int32, _L)

    @plsc.parallel_loop(0, _NG, unroll=4)
    def pa(g):
        h = hist[pl.ds(g * _L, _L)]
        hist[pl.ds(g * _L, _L)] = zeros_i
        for u in range(1, nsub):
            h = h + hist[pl.ds(u * _NB + g * _L, _L)]
            hist[pl.ds(u * _NB + g * _L, _L)] = zeros_i
        cums[pl.ds(g * _L, _L)] = plsc.cumsum(h)

    def pb(j, tot):
        gt = plsc.load_gather(cums, [(j * _L + iota) * _L + (_L - 1)])
        cs = plsc.cumsum(gt) + tot
        gcums[pl.ds(j * _L, _L)] = cs
        return jnp.max(cs)

    lax.fori_loop(0, _NG // _L, pb, jnp.int32(0))

    kt = kk + 1
    c1 = plsc.load_gather(gcums, [iota * 8 + 7])
    s1 = plsc.all_reduce_ffs(c1 >= kt)
    c2 = plsc.load_gather(gcums, [jnp.minimum(s1 * 8 + iota, _NG - 1)])
    s2 = plsc.all_reduce_ffs(c2 >= kt)
    g = s1 * 8 + s2
    gb = plsc.load_gather(gcums, [jnp.maximum(g - 1, 0)])
    base = jnp.where(g >= 1, gb, 0)
    cf = plsc.load_gather(cums, [g * _L + iota]) + base
    f = plsc.all_reduce_ffs(cf >= kt)
    b = g * _L + f
    lb = plsc.load_gather(cums, [jnp.maximum(b - 1, g * _L)])
    cum_before = jnp.where(f >= 1, lb + base, base)
    return b, kk - cum_before


def _pass1(row_v, hist):
    ones_i = jnp.ones((_L,), jnp.int32)

    @plsc.parallel_loop(0, _NV, unroll=_U, carry=jnp.zeros((_L,), jnp.float32))
    def p1(i, acc):
        v = row_v[pl.ds(i * _L, _L)]
        d1 = plsc.bitcast(v, jnp.int32) >> 19
        plsc.addupdate_scatter(hist, [((i & (_U - 1)) << _B) + d1], ones_i)
        return acc + v

    return jnp.sum(p1) * (1.0 / _N)


def _pass2(row_v, b1, hist):
    ones_i = jnp.ones((_L,), jnp.int32)

    @plsc.parallel_loop(0, _NV, unroll=_U)
    def p2(i):
        bits = plsc.bitcast(row_v[pl.ds(i * _L, _L)], jnp.int32)
        m = (bits >> 19) == b1
        plsc.addupdate_scatter(
            hist, [((i & (_U - 1)) << _B) + ((bits >> 8) & (_NB - 1))],
            ones_i, mask=m)


def _sc_body(x_hbm, out_hbm, row_a, row_b, hist, cums, gcums, out_v,
             sem_a, sem_b):
    wid = lax.axis_index("s") * _NC + lax.axis_index("c")
    iota = lax.iota(jnp.int32, _L)
    zeros_i = jnp.zeros((_L,), jnp.int32)

    rows = [row_a, row_b]
    sems = [sem_a, sem_b]
    base = wid * _RPW
    copies = [None, None]
    copies[0] = pltpu.async_copy(x_hbm.at[base], row_a, sem_a)

    @plsc.parallel_loop(0, _U * _NG, unroll=8)
    def z(g):
        hist[pl.ds(g * _L, _L)] = zeros_i

    kk0 = jnp.full((_L,), _K, jnp.int32)
    out_acc = jnp.zeros((_L,), jnp.float32)
    for j in range(_RPW):
        copies[j % 2].wait()
        if j + 1 < _RPW:
            copies[(j + 1) % 2] = pltpu.async_copy(
                x_hbm.at[base + j + 1], rows[(j + 1) % 2], sems[(j + 1) % 2])
        mean = _pass1(rows[j % 2], hist)
        b1, kk1 = _scan_select(hist, cums, gcums, kk0, _U)
        _pass2(rows[j % 2], b1, hist)
        b2, _ = _scan_select(hist, cums, gcums, kk1, _U)
        m_key = (b1 << 19) | (b2 << 8) | 128
        m_bit = plsc.bitcast(m_key, jnp.float32)
        margin = jnp.abs(m_bit - 0.5)
        delta = mean * margin
        rep = jnp.where(m_bit > 0.5, 0.5 + delta, m_bit + delta)
        out_acc = jnp.where(iota == j, rep, out_acc)

    out_v[...] = out_acc
    pltpu.sync_copy(out_v, out_hbm.at[wid])


@functools.cache
def _build():
    mesh = plsc.VectorSubcoreMesh(core_axis_name="c", subcore_axis_name="s")
    return functools.partial(
        pl.kernel,
        out_type=jax.ShapeDtypeStruct((_NW, _L), jnp.float32),
        mesh=mesh,
        scratch_types=[
            pltpu.VMEM((_N,), jnp.float32),
            pltpu.VMEM((_N,), jnp.float32),
            pltpu.VMEM((_U * _NB,), jnp.int32),
            pltpu.VMEM((_NB,), jnp.int32),
            pltpu.VMEM((_NG,), jnp.int32),
            pltpu.VMEM((_L,), jnp.float32),
            pltpu.SemaphoreType.DMA,
            pltpu.SemaphoreType.DMA,
        ],
        compiler_params=pltpu.CompilerParams(needs_layout_passes=False),
    )(_sc_body)


@jax.jit
def kernel(x):
    out2d = _build()(x)
    return out2d[:, :_RPW].reshape(-1)
